# baseline (speedup 1.0000x reference)
import functools

import jax
import jax.numpy as jnp
from jax import lax
from jax.experimental import pallas as pl
from jax.experimental.pallas import tpu as pltpu
from jax.experimental.pallas import tpu_sc as plsc

NN = 10000
NE = 320000
HD = 128
NL = 4
NG = 64
OUTD = 128

NWORK = 32
K = 128
CHUNKS = 81
CH_SC0 = 81
CH_SC1 = 2 * CHUNKS - CH_SC0
EW = K * CHUNKS
E_PAD = NWORK * EW
CH_TOT = NWORK * CHUNKS
TREP = 128
N_ACC = 10112
RPT = N_ACC // 16
RCH = (128, 128, 128, 128, 120)
JUNK = NN + 8

_HIGH = lax.Precision.HIGHEST


def _sc_segment_sum(feat=HD):
  mesh = plsc.VectorSubcoreMesh(core_axis_name="c", subcore_axis_name="s")
  scratch = (
      [pltpu.VMEM((2, K), jnp.int32) for _ in range(3)] +
      [pltpu.VMEM((K, feat), jnp.float32) for _ in range(3)] +
      [pltpu.VMEM_SHARED((N_ACC, feat), jnp.float32)] +
      [pltpu.SemaphoreType.DMA for _ in range(6)])

  def entry(table_hbm, idx_hbm, zero_hbm, out_hbm, *refs):
    idx = refs[0:3]
    rows = refs[3:6]
    acc = refs[6]
    sg = refs[7:10]
    si = refs[10:13]
    core = lax.axis_index("c")
    sub = lax.axis_index("s")
    rb = sub * RPT
    c0 = jnp.where(core == 0, sub * CH_SC0, 16 * CH_SC0 + sub * CH_SC1)
    n_trip = jnp.where(core == 0, CH_SC0 // 3, CH_SC1 // 3)
    pltpu.sync_copy(zero_hbm, rows[0])
    off = 0
    for w in RCH:
      pltpu.sync_copy(rows[0].at[pl.ds(0, w)], acc.at[pl.ds(rb + off, w)])
      off += w
    for u in range(3):
      pltpu.async_copy(idx_hbm.at[c0 + u], idx[u], si[u])
    for u in range(2):
      pltpu.make_async_copy(idx_hbm.at[c0 + u], idx[u], si[u]).wait()
      pltpu.async_copy(table_hbm.at[idx[u].at[0]], rows[u], sg[u])
    plsc.subcore_barrier()

    def triple(i, carry):
      for u in range(3):
        un = (u + 2) % 3
        c = i * 3 + u
        pltpu.make_async_copy(table_hbm.at[idx[u].at[0]], rows[u],
                              sg[u]).wait()
        pltpu.sync_copy(rows[u], acc.at[idx[u].at[1]], add=True)
        pltpu.async_copy(idx_hbm.at[c0 + c + 3], idx[u], si[u])
        pltpu.make_async_copy(idx_hbm.at[c0 + c + 2], idx[un], si[un]).wait()
        pltpu.async_copy(table_hbm.at[idx[un].at[0]], rows[un], sg[un])
      return carry

    lax.fori_loop(0, n_trip, triple, 0)
    for u in range(2):
      pltpu.make_async_copy(table_hbm.at[idx[u].at[0]], rows[u], sg[u]).wait()
    pltpu.make_async_copy(idx_hbm.at[c0], idx[2], si[2]).wait()
    plsc.subcore_barrier()
    o = pl.multiple_of(rb, 8)
    pltpu.sync_copy(acc.at[pl.ds(o, RPT)], out_hbm.at[core, pl.ds(o, RPT)])

  params = {}
  if feat != HD:
    params["compiler_params"] = pltpu.CompilerParams(use_tc_tiling_on_sc=False)
  return functools.partial(
      pl.kernel, mesh=mesh,
      out_type=jax.ShapeDtypeStruct((2, N_ACC, feat), jnp.float32),
      scratch_types=scratch, **params)(entry)


_sc_gather_add = _sc_segment_sum()
_sc_gather_add16 = _sc_segment_sum(16)


def _chunk_idx(srcs, dsts):
  srcr = srcs.reshape(CH_TOT, K)
  dstr = dsts.reshape(CH_TOT, K)
  packed = jnp.stack([srcr, dstr], axis=1)
  pad_src = (jnp.arange(4 * K, dtype=jnp.int32) % 1024).reshape(4, K)
  pad = jnp.stack([pad_src, jnp.full((4, K), JUNK, jnp.int32)], axis=1)
  return jnp.concatenate([packed, pad], axis=0)


def _tc_encode(xf, da, t0):
  def body(x_ref, d_ref, t_ref, o_ref):
    o_ref[...] = jnp.dot(x_ref[...], d_ref[...], precision=_HIGH,
                         preferred_element_type=jnp.float32) + t_ref[...]
  return pl.pallas_call(
      body, out_shape=jax.ShapeDtypeStruct((NN, HD), jnp.float32))(xf, da, t0)


def _layer_core(p_ref, a_ref, h_ref, dl_ref, b0_ref, w_ref, bi_ref, g_ref,
                be_ref):
  aa = a_ref[0, :NN, :] + a_ref[1, :NN, :]
  bond = jnp.dot(aa, dl_ref[...], precision=_HIGH,
                 preferred_element_type=jnp.float32) + b0_ref[...]
  agg = p_ref[0, :NN, :] + p_ref[1, :NN, :] + h_ref[...] + bond
  z = jnp.dot(agg, w_ref[...],
              preferred_element_type=jnp.float32) + bi_ref[...]
  mu = jnp.mean(z, axis=0, keepdims=True)
  var = jnp.mean(jnp.square(z - mu), axis=0, keepdims=True)
  zn = (z - mu) * lax.rsqrt(var + 1e-5) * g_ref[...] + be_ref[...]
  return jnp.maximum(zn, 0.0)


def _tc_layer(p, dA, h, dl, b0, w, bias, gam, bet):
  def body(p_ref, a_ref, h_ref, dl_ref, b0_ref, w_ref, bi_ref, g_ref, be_ref,
           o_ref):
    o_ref[...] = _layer_core(p_ref, a_ref, h_ref, dl_ref, b0_ref, w_ref,
                             bi_ref, g_ref, be_ref)
  return pl.pallas_call(
      body, out_shape=jax.ShapeDtypeStruct((NN, HD), jnp.float32))(
          p, dA, h, dl, b0, w, bias, gam, bet)


def _tc_final(p, dA, h, dl, b0, w, bias, gam, bet, batch_row, w1, b1r, w2,
              b2r):
  def body(p_ref, a_ref, h_ref, dl_ref, b0_ref, w_ref, bi_ref, g_ref, be_ref,
           bt_ref, w1_ref, b1_ref, w2_ref, b2_ref, h_out, pred_out):
    h4 = _layer_core(p_ref, a_ref, h_ref, dl_ref, b0_ref, w_ref, bi_ref,
                     g_ref, be_ref)
    h_out[...] = h4
    gid = lax.broadcasted_iota(jnp.int32, (NG, 1), 0)
    m = (bt_ref[...] == gid).astype(jnp.float32)
    gsum = jnp.dot(m, h4, preferred_element_type=jnp.float32)
    cnt = jnp.sum(m, axis=1, keepdims=True)
    gmean = gsum / jnp.maximum(cnt, 1.0)
    act = jnp.maximum(
        jnp.dot(gmean, w1_ref[...],
                preferred_element_type=jnp.float32) + b1_ref[...], 0.0)
    pred_out[...] = jnp.dot(act, w2_ref[...],
                            preferred_element_type=jnp.float32) + b2_ref[...]

  return pl.pallas_call(
      body, out_shape=(jax.ShapeDtypeStruct((NN, HD), jnp.float32),
                       jax.ShapeDtypeStruct((NG, OUTD), jnp.float32)))(
          p, dA, h, dl, b0, w, bias, gam, bet, batch_row, w1, b1r, w2, b2r)


def kernel(x, edge_index, edge_attr, batch, atom_emb, bond_emb, W, b, gamma,
           beta, W1, b1, W2, b2):
  f32 = jnp.float32
  pad = E_PAD - NE
  parange = jnp.arange(pad, dtype=jnp.int32)
  src_p = jnp.concatenate([edge_index[0].astype(jnp.int32), parange % NN])
  dst_p = jnp.concatenate([edge_index[1].astype(jnp.int32),
                           JUNK + parange % 100])
  edge_idx = _chunk_idx(src_p, dst_p)
  code = jnp.sum(edge_attr.astype(jnp.int32) * (2 ** jnp.arange(5))[None, :],
                 axis=1)
  code_p = jnp.concatenate([code, jnp.zeros((pad,), jnp.int32)])
  code_p = code_p + 32 * (jnp.arange(E_PAD, dtype=jnp.int32) % TREP)
  attr_idx = _chunk_idx(code_p, dst_p)
  cbits = ((jnp.arange(32)[:, None] >> jnp.arange(5)[None, :]) & 1).astype(f32)
  tind = jnp.concatenate(
      [jnp.ones((32, 1), f32), cbits, jnp.zeros((32, 16 - 6), f32)], axis=1)
  tind = jnp.tile(tind, (TREP, 1))
  zero_h = jnp.zeros((K, HD), f32)
  zero_16 = jnp.zeros((K, 16), f32)

  xf = jnp.concatenate([x.astype(f32), jnp.zeros((NN, 16 - 9), f32)], axis=1)
  da = jnp.concatenate([atom_emb[:, 1, :] - atom_emb[:, 0, :],
                        jnp.zeros((16 - 9, HD), f32)], axis=0)
  t0 = jnp.sum(atom_emb[:, 0, :], axis=0, keepdims=True)

  b0 = jnp.sum(bond_emb[:, :, 0, :], axis=1)
  db = bond_emb[:, :, 1, :] - bond_emb[:, :, 0, :]
  dl_all = jnp.concatenate(
      [b0[:, None, :], db, jnp.zeros((NL, 16 - 6, HD), f32)], axis=1)

  batch_row = batch.astype(jnp.int32).reshape(1, NN)
  brow = b.reshape(NL, 1, HD)
  grow = gamma.reshape(NL, 1, HD)
  berow = beta.reshape(NL, 1, HD)
  b1r = b1.reshape(1, -1)
  b2r = b2.reshape(1, -1)

  h = _tc_encode(xf, da, t0)
  pred = None
  dA = None
  for l in range(NL):
    p = _sc_gather_add(h, edge_idx, zero_h)
    if l == 0:
      dA = _sc_gather_add16(tind, attr_idx, zero_16 + p[0, :1, :16] * 0.0)
    if l < NL - 1:
      h = _tc_layer(p, dA, h, dl_all[l], b0[l:l + 1], W[l], brow[l], grow[l],
                    berow[l])
    else:
      h, pred = _tc_final(p, dA, h, dl_all[l], b0[l:l + 1], W[l], brow[l],
                          grow[l], berow[l], batch_row, W1, b1r, W2, b2r)
  return (pred, h)


# speedup vs baseline: 1.0367x; 1.0367x over previous
import functools

import jax
import jax.numpy as jnp
from jax import lax
from jax.experimental import pallas as pl
from jax.experimental.pallas import tpu as pltpu
from jax.experimental.pallas import tpu_sc as plsc

NN = 10000
NE = 320000
HD = 128
NL = 4
NG = 64
OUTD = 128

NWORK = 32
K = 128
CHUNKS = 80
CH_SC0 = 80
CH_SC1 = 2 * CHUNKS - CH_SC0
EW = K * CHUNKS
E_PAD = NWORK * EW
CH_TOT = NWORK * CHUNKS
TREP = 128
N_ACC = 10112
RPT = N_ACC // 16
RCH = (128, 128, 128, 128, 120)
JUNK = NN + 8

_HIGH = lax.Precision.HIGHEST


def _sc_segment_sum(feat=HD):
  mesh = plsc.VectorSubcoreMesh(core_axis_name="c", subcore_axis_name="s")
  scratch = (
      [pltpu.VMEM((2, K), jnp.int32) for _ in range(4)] +
      [pltpu.VMEM((K, feat), jnp.float32) for _ in range(2)] +
      [pltpu.VMEM((K, feat), jnp.float32),
       pltpu.VMEM_SHARED((N_ACC, feat), jnp.float32)] +
      [pltpu.SemaphoreType.DMA for _ in range(6)])

  def entry(table_hbm, idx_hbm, zero_hbm, out_hbm, *refs):
    idx = refs[0:4]
    rows = refs[4:6]
    zbuf = refs[6]
    acc = refs[7]
    sg = refs[8:10]
    si = refs[10:14]
    core = lax.axis_index("c")
    sub = lax.axis_index("s")
    rb = sub * RPT
    c0 = jnp.where(core == 0, sub * CH_SC0, 16 * CH_SC0 + sub * CH_SC1)
    n_quads = jnp.where(core == 0, CH_SC0 // 4, CH_SC1 // 4)
    for u in range(4):
      pltpu.async_copy(idx_hbm.at[c0 + u], idx[u], si[u])
    for u in range(2):
      pltpu.make_async_copy(idx_hbm.at[c0 + u], idx[u], si[u]).wait()
      pltpu.async_copy(table_hbm.at[idx[u].at[0]], rows[u], sg[u])
    pltpu.sync_copy(zero_hbm, zbuf)
    off = 0
    for w in RCH:
      pltpu.sync_copy(zbuf.at[pl.ds(0, w)], acc.at[pl.ds(rb + off, w)])
      off += w
    plsc.subcore_barrier()

    def quad(i, carry):
      for u in range(4):
        b, q, qn = u % 2, u, (u + 2) % 4
        c = i * 4 + u
        pltpu.make_async_copy(table_hbm.at[idx[q].at[0]], rows[b],
                              sg[b]).wait()
        pltpu.sync_copy(rows[b], acc.at[idx[q].at[1]], add=True)
        pltpu.async_copy(idx_hbm.at[c0 + c + 4], idx[q], si[q])
        pltpu.make_async_copy(idx_hbm.at[c0 + c + 2], idx[qn], si[qn]).wait()
        pltpu.async_copy(table_hbm.at[idx[qn].at[0]], rows[b], sg[b])
      return carry

    lax.fori_loop(0, n_quads, quad, 0)
    for u in range(2):
      pltpu.make_async_copy(table_hbm.at[idx[u].at[0]], rows[u], sg[u]).wait()
    for q in range(2, 4):
      pltpu.make_async_copy(idx_hbm.at[c0 + q], idx[q], si[q]).wait()
    plsc.subcore_barrier()
    o = pl.multiple_of(rb, 8)
    pltpu.sync_copy(acc.at[pl.ds(o, RPT)], out_hbm.at[core, pl.ds(o, RPT)])

  params = {}
  if feat != HD:
    params["compiler_params"] = pltpu.CompilerParams(use_tc_tiling_on_sc=False)
  return functools.partial(
      pl.kernel, mesh=mesh,
      out_type=jax.ShapeDtypeStruct((2, N_ACC, feat), jnp.float32),
      scratch_types=scratch, **params)(entry)


_sc_gather_add = _sc_segment_sum()
_sc_gather_add16 = _sc_segment_sum(16)


def _chunk_idx(srcs, dsts):
  srcr = srcs.reshape(CH_TOT, K)
  dstr = dsts.reshape(CH_TOT, K)
  packed = jnp.stack([srcr, dstr], axis=1)
  pad_src = (jnp.arange(4 * K, dtype=jnp.int32) % 1024).reshape(4, K)
  pad = jnp.stack([pad_src, jnp.full((4, K), JUNK, jnp.int32)], axis=1)
  return jnp.concatenate([packed, pad], axis=0)


def _tc_encode(xf, da, t0):
  def body(x_ref, d_ref, t_ref, o_ref):
    o_ref[...] = jnp.dot(x_ref[...], d_ref[...], precision=_HIGH,
                         preferred_element_type=jnp.float32) + t_ref[...]
  return pl.pallas_call(
      body, out_shape=jax.ShapeDtypeStruct((NN, HD), jnp.float32))(xf, da, t0)


def _layer_core(p_ref, a_ref, h_ref, dl_ref, b0_ref, w_ref, bi_ref, g_ref,
                be_ref):
  aa = a_ref[0, :NN, :] + a_ref[1, :NN, :]
  bond = jnp.dot(aa, dl_ref[...], precision=_HIGH,
                 preferred_element_type=jnp.float32) + b0_ref[...]
  agg = p_ref[0, :NN, :] + p_ref[1, :NN, :] + h_ref[...] + bond
  z = jnp.dot(agg, w_ref[...],
              preferred_element_type=jnp.float32) + bi_ref[...]
  mu = jnp.mean(z, axis=0, keepdims=True)
  var = jnp.mean(jnp.square(z - mu), axis=0, keepdims=True)
  zn = (z - mu) * lax.rsqrt(var + 1e-5) * g_ref[...] + be_ref[...]
  return jnp.maximum(zn, 0.0)


def _tc_layer(p, dA, h, dl, b0, w, bias, gam, bet):
  def body(p_ref, a_ref, h_ref, dl_ref, b0_ref, w_ref, bi_ref, g_ref, be_ref,
           o_ref):
    o_ref[...] = _layer_core(p_ref, a_ref, h_ref, dl_ref, b0_ref, w_ref,
                             bi_ref, g_ref, be_ref)
  return pl.pallas_call(
      body, out_shape=jax.ShapeDtypeStruct((NN, HD), jnp.float32))(
          p, dA, h, dl, b0, w, bias, gam, bet)


def _tc_final(p, dA, h, dl, b0, w, bias, gam, bet, batch_row, w1, b1r, w2,
              b2r):
  def body(p_ref, a_ref, h_ref, dl_ref, b0_ref, w_ref, bi_ref, g_ref, be_ref,
           bt_ref, w1_ref, b1_ref, w2_ref, b2_ref, h_out, pred_out):
    h4 = _layer_core(p_ref, a_ref, h_ref, dl_ref, b0_ref, w_ref, bi_ref,
                     g_ref, be_ref)
    h_out[...] = h4
    gid = lax.broadcasted_iota(jnp.int32, (NG, 1), 0)
    m = (bt_ref[...] == gid).astype(jnp.float32)
    gsum = jnp.dot(m, h4, preferred_element_type=jnp.float32)
    cnt = jnp.sum(m, axis=1, keepdims=True)
    gmean = gsum / jnp.maximum(cnt, 1.0)
    act = jnp.maximum(
        jnp.dot(gmean, w1_ref[...],
                preferred_element_type=jnp.float32) + b1_ref[...], 0.0)
    pred_out[...] = jnp.dot(act, w2_ref[...],
                            preferred_element_type=jnp.float32) + b2_ref[...]

  return pl.pallas_call(
      body, out_shape=(jax.ShapeDtypeStruct((NN, HD), jnp.float32),
                       jax.ShapeDtypeStruct((NG, OUTD), jnp.float32)))(
          p, dA, h, dl, b0, w, bias, gam, bet, batch_row, w1, b1r, w2, b2r)


def kernel(x, edge_index, edge_attr, batch, atom_emb, bond_emb, W, b, gamma,
           beta, W1, b1, W2, b2):
  f32 = jnp.float32
  pad = E_PAD - NE
  parange = jnp.arange(pad, dtype=jnp.int32)
  src_p = jnp.concatenate([edge_index[0].astype(jnp.int32), parange % NN])
  dst_p = jnp.concatenate([edge_index[1].astype(jnp.int32),
                           JUNK + parange % 100])
  edge_idx = _chunk_idx(src_p, dst_p)
  code = jnp.sum(edge_attr.astype(jnp.int32) * (2 ** jnp.arange(5))[None, :],
                 axis=1)
  code_p = jnp.concatenate([code, jnp.zeros((pad,), jnp.int32)])
  code_p = code_p + 32 * (jnp.arange(E_PAD, dtype=jnp.int32) % TREP)
  attr_idx = _chunk_idx(code_p, dst_p)
  cbits = ((jnp.arange(32)[:, None] >> jnp.arange(5)[None, :]) & 1).astype(f32)
  tind = jnp.concatenate(
      [jnp.ones((32, 1), f32), cbits, jnp.zeros((32, 16 - 6), f32)], axis=1)
  tind = jnp.tile(tind, (TREP, 1))
  zero_h = jnp.zeros((K, HD), f32)
  zero_16 = jnp.zeros((K, 16), f32)

  xf = jnp.concatenate([x.astype(f32), jnp.zeros((NN, 16 - 9), f32)], axis=1)
  da = jnp.concatenate([atom_emb[:, 1, :] - atom_emb[:, 0, :],
                        jnp.zeros((16 - 9, HD), f32)], axis=0)
  t0 = jnp.sum(atom_emb[:, 0, :], axis=0, keepdims=True)

  b0 = jnp.sum(bond_emb[:, :, 0, :], axis=1)
  db = bond_emb[:, :, 1, :] - bond_emb[:, :, 0, :]
  dl_all = jnp.concatenate(
      [b0[:, None, :], db, jnp.zeros((NL, 16 - 6, HD), f32)], axis=1)

  batch_row = batch.astype(jnp.int32).reshape(1, NN)
  brow = b.reshape(NL, 1, HD)
  grow = gamma.reshape(NL, 1, HD)
  berow = beta.reshape(NL, 1, HD)
  b1r = b1.reshape(1, -1)
  b2r = b2.reshape(1, -1)

  dA = _sc_gather_add16(tind, attr_idx, zero_16)
  h = _tc_encode(xf, da, t0)
  pred = None
  for l in range(NL):
    p = _sc_gather_add(h, edge_idx, zero_h)
    if l < NL - 1:
      h = _tc_layer(p, dA, h, dl_all[l], b0[l:l + 1], W[l], brow[l], grow[l],
                    berow[l])
    else:
      h, pred = _tc_final(p, dA, h, dl_all[l], b0[l:l + 1], W[l], brow[l],
                          grow[l], berow[l], batch_row, W1, b1r, W2, b2r)
  return (pred, h)
